# Initial kernel scaffold; baseline (speedup 1.0000x reference)
import jax
import jax.numpy as jnp
from jax.experimental import pallas as pl


def kernel(x, W_embed, b_embed, Wg, We, be, Wr, br, Wc, bc, Wp, bp):
    raise NotImplementedError("write your pallas kernel here")


# speedup vs baseline: 7.4895x; 7.4895x over previous
import functools

import jax
import jax.numpy as jnp
from jax.experimental import pallas as pl

N = 32768
D_IN, D_MODEL, E, D_OUT = 4, 16, 8, 4
BLOCK = 4096


def _moe_block_kernel(x_ref, We_ref, Wemb_ref, bemb_ref, Wg_ref, be_ref,
                      Wr_ref, br_ref, Wc_ref, bc_ref, Wp_ref, bp_ref,
                      out_ref):
    f32 = jnp.float32
    x = x_ref[...]
    h = jnp.dot(x, Wemb_ref[...], preferred_element_type=f32) + bemb_ref[...]
    logits = jnp.dot(h, Wg_ref[...], preferred_element_type=f32)
    m = jnp.max(logits, axis=-1, keepdims=True)
    ex = jnp.exp(logits - m)
    denom = jnp.sum(ex, axis=-1, keepdims=True)
    gate = jnp.max(ex, axis=-1, keepdims=True) / denom
    iota = jax.lax.broadcasted_iota(jnp.int32, logits.shape, 1)
    ismax = logits >= m
    idx = jnp.min(jnp.where(ismax, iota, E), axis=-1, keepdims=True)
    onehot = (iota == idx).astype(f32)

    acc = jnp.zeros((x.shape[0], D_MODEL), f32)
    for e in range(E):
        eo = jnp.dot(h, We_ref[e], preferred_element_type=f32) + be_ref[e]
        acc = acc + onehot[:, e:e + 1] * eo
    moe = gate * acc

    res = jnp.dot(h, Wr_ref[...], preferred_element_type=f32) + br_ref[...]
    c = jnp.dot(h, Wc_ref[...], preferred_element_type=f32) + bc_ref[...]
    cm = jnp.max(c, axis=-1, keepdims=True)
    cex = jnp.exp(c - cm)
    coef = cex / jnp.sum(cex, axis=-1, keepdims=True)

    comb = moe * coef[:, 0:1] + res * coef[:, 1:2]
    out_ref[...] = (jnp.dot(comb, Wp_ref[...], preferred_element_type=f32)
                    + bp_ref[...])


@jax.jit
def kernel(x, W_embed, b_embed, Wg, We, be, Wr, br, Wc, bc, Wp, bp):
    grid = (N // BLOCK,)

    def full(shape):
        return pl.BlockSpec(shape, lambda i: tuple(0 for _ in shape))

    out = pl.pallas_call(
        _moe_block_kernel,
        grid=grid,
        in_specs=[
            pl.BlockSpec((BLOCK, D_IN), lambda i: (i, 0)),
            full((E, D_MODEL, D_MODEL)),
            full((D_IN, D_MODEL)),
            full((1, D_MODEL)),
            full((D_MODEL, E)),
            full((E, D_MODEL)),
            full((D_MODEL, D_MODEL)),
            full((1, D_MODEL)),
            full((D_MODEL, 2)),
            full((1, 2)),
            full((D_MODEL, D_OUT)),
            full((1, D_OUT)),
        ],
        out_specs=pl.BlockSpec((BLOCK, D_OUT), lambda i: (i, 0)),
        out_shape=jax.ShapeDtypeStruct((N, D_OUT), jnp.float32),
    )(x, We, W_embed, b_embed.reshape(1, -1), Wg, be, Wr,
      br.reshape(1, -1), Wc, bc.reshape(1, -1), Wp, bp.reshape(1, -1))
    return out


# speedup vs baseline: 15.5912x; 2.0817x over previous
import jax
import jax.numpy as jnp
from jax import lax
from jax.experimental import pallas as pl

N = 32768
D_IN, D_MODEL, E, D_OUT = 4, 16, 8, 4
BLOCK = 4096

_DN_RR = (((1,), (1,)), ((), ()))
_DN_CC = (((0,), (0,)), ((), ()))


def _moe_kernel(x_ref, WembT_ref, bemb_ref, WgT_ref, WeAllT_ref, beAll_ref,
                WrT_ref, br_ref, WcT_ref, bc_ref, Wp_ref, bp_ref, out_ref):
    f32 = jnp.float32
    x = x_ref[...]
    hT = lax.dot_general(WembT_ref[...], x, _DN_RR,
                         preferred_element_type=f32) + bemb_ref[...]
    logits = jnp.dot(WgT_ref[...], hT, preferred_element_type=f32)
    m = jnp.max(logits, axis=0, keepdims=True)
    ex = jnp.exp(logits - m)
    denom = jnp.sum(ex, axis=0, keepdims=True)
    gate = jnp.max(ex, axis=0, keepdims=True) / denom
    iota = lax.broadcasted_iota(jnp.int32, logits.shape, 0)
    ismax = logits >= m
    idx = jnp.min(jnp.where(ismax, iota, E), axis=0, keepdims=True)
    onehot = (iota == idx).astype(f32)

    EO = jnp.dot(WeAllT_ref[...], hT, preferred_element_type=f32)
    EO = EO + beAll_ref[...]
    acc = jnp.zeros(hT.shape, f32)
    for e in range(E):
        acc = acc + EO[e * D_MODEL:(e + 1) * D_MODEL, :] * onehot[e:e + 1, :]
    moe = gate * acc

    res = jnp.dot(WrT_ref[...], hT, preferred_element_type=f32) + br_ref[...]
    c = jnp.dot(WcT_ref[...], hT, preferred_element_type=f32) + bc_ref[...]
    cm = jnp.max(c, axis=0, keepdims=True)
    cex = jnp.exp(c - cm)
    coef = cex / jnp.sum(cex, axis=0, keepdims=True)

    comb = moe * coef[0:1, :] + res * coef[1:2, :]
    out_ref[...] = (lax.dot_general(comb, Wp_ref[...], _DN_CC,
                                    preferred_element_type=f32)
                    + bp_ref[...])


@jax.jit
def kernel(x, W_embed, b_embed, Wg, We, be, Wr, br, Wc, bc, Wp, bp):
    grid = (N // BLOCK,)

    def full(shape):
        return pl.BlockSpec(shape, lambda i: tuple(0 for _ in shape))

    out = pl.pallas_call(
        _moe_kernel,
        grid=grid,
        in_specs=[
            pl.BlockSpec((BLOCK, D_IN), lambda i: (i, 0)),
            full((D_MODEL, D_IN)),
            full((D_MODEL, 1)),
            full((E, D_MODEL)),
            full((E * D_MODEL, D_MODEL)),
            full((E * D_MODEL, 1)),
            full((D_MODEL, D_MODEL)),
            full((D_MODEL, 1)),
            full((2, D_MODEL)),
            full((2, 1)),
            full((D_MODEL, D_OUT)),
            full((1, D_OUT)),
        ],
        out_specs=pl.BlockSpec((BLOCK, D_OUT), lambda i: (i, 0)),
        out_shape=jax.ShapeDtypeStruct((N, D_OUT), jnp.float32),
    )(x,
      W_embed.T, b_embed.reshape(-1, 1),
      Wg.T,
      We.transpose(0, 2, 1).reshape(E * D_MODEL, D_MODEL),
      be.reshape(-1, 1),
      Wr.T, br.reshape(-1, 1),
      Wc.T, bc.reshape(-1, 1),
      Wp, bp.reshape(1, -1))
    return out


# speedup vs baseline: 16.7656x; 1.0753x over previous
import jax
import jax.numpy as jnp
from jax import lax
from jax.experimental import pallas as pl

N = 32768
D_IN, D_MODEL, E, D_OUT = 4, 16, 8, 4
BLOCK = 8192

_DN_00 = (((0,), (0,)), ((), ()))
_DN_01 = (((0,), (1,)), ((), ()))


def _moe_kernel(x_ref, Wemb_ref, bemb_ref, Wg_ref, We_ref, beAll_ref,
                Wr_ref, br_ref, Wc_ref, bc_ref, Wp_ref, bp_ref, out_ref):
    f32 = jnp.float32
    x = x_ref[...]
    hT = lax.dot_general(Wemb_ref[...], x, _DN_01,
                         preferred_element_type=f32) + bemb_ref[...]
    logits = lax.dot_general(Wg_ref[...], hT, _DN_00,
                             preferred_element_type=f32)
    m = jnp.max(logits, axis=0, keepdims=True)
    ex = jnp.exp(logits - m)
    denom = jnp.sum(ex, axis=0, keepdims=True)
    gate = jnp.max(ex, axis=0, keepdims=True) / denom
    iota = lax.broadcasted_iota(jnp.int32, logits.shape, 0)
    ismax = logits >= m
    idx = jnp.min(jnp.where(ismax, iota, E), axis=0, keepdims=True)
    onehot = (iota == idx).astype(f32)

    WeAllT = jnp.transpose(We_ref[...], (0, 2, 1)).reshape(
        E * D_MODEL, D_MODEL)
    EO = jnp.dot(WeAllT, hT, preferred_element_type=f32)
    EO = EO + beAll_ref[...]
    acc = jnp.zeros(hT.shape, f32)
    for e in range(E):
        acc = acc + EO[e * D_MODEL:(e + 1) * D_MODEL, :] * onehot[e:e + 1, :]
    moe = gate * acc

    res = lax.dot_general(Wr_ref[...], hT, _DN_00,
                          preferred_element_type=f32) + br_ref[...]
    c = lax.dot_general(Wc_ref[...], hT, _DN_00,
                        preferred_element_type=f32) + bc_ref[...]
    cm = jnp.max(c, axis=0, keepdims=True)
    cex = jnp.exp(c - cm)
    coef = cex / jnp.sum(cex, axis=0, keepdims=True)

    comb = moe * coef[0:1, :] + res * coef[1:2, :]
    out_ref[...] = (lax.dot_general(comb, Wp_ref[...], _DN_00,
                                    preferred_element_type=f32)
                    + bp_ref[...])


@jax.jit
def kernel(x, W_embed, b_embed, Wg, We, be, Wr, br, Wc, bc, Wp, bp):
    grid = (N // BLOCK,)

    def full(shape):
        return pl.BlockSpec(shape, lambda i: tuple(0 for _ in shape))

    out = pl.pallas_call(
        _moe_kernel,
        grid=grid,
        in_specs=[
            pl.BlockSpec((BLOCK, D_IN), lambda i: (i, 0)),
            full((D_IN, D_MODEL)),
            full((D_MODEL, 1)),
            full((D_MODEL, E)),
            full((E, D_MODEL, D_MODEL)),
            full((E * D_MODEL, 1)),
            full((D_MODEL, D_MODEL)),
            full((D_MODEL, 1)),
            full((D_MODEL, 2)),
            full((2, 1)),
            full((D_MODEL, D_OUT)),
            full((1, D_OUT)),
        ],
        out_specs=pl.BlockSpec((BLOCK, D_OUT), lambda i: (i, 0)),
        out_shape=jax.ShapeDtypeStruct((N, D_OUT), jnp.float32),
    )(x,
      W_embed, b_embed.reshape(-1, 1),
      Wg,
      We,
      be.reshape(-1, 1),
      Wr, br.reshape(-1, 1),
      Wc, bc.reshape(-1, 1),
      Wp, bp.reshape(1, -1))
    return out
